# baseline (speedup 1.0000x reference)
import jax
import jax.numpy as jnp
from jax.experimental import pallas as pl
from jax.experimental.pallas import tpu as pltpu


def _add_noise_block(t_ref, sa_ref, sb_ref, x0_ref, noise_ref, out_ref):
    n = pl.program_id(0)
    tt = t_ref[n]
    a = sa_ref[tt]
    b = sb_ref[tt]
    out_ref[...] = a * x0_ref[...] + b * noise_ref[...]


def kernel(x0, noise, t, sqrt_alphas_cumprod, sqrt_one_minus_alphas_cumprod):
    n, c, h, w = x0.shape
    rows = c * h
    x2 = x0.reshape(n * rows, w)
    n2 = noise.reshape(n * rows, w)

    out = pl.pallas_call(
        _add_noise_block,
        grid_spec=pltpu.PrefetchScalarGridSpec(
            num_scalar_prefetch=3,
            grid=(n,),
            in_specs=[
                pl.BlockSpec((rows, w), lambda i, *_: (i, 0)),
                pl.BlockSpec((rows, w), lambda i, *_: (i, 0)),
            ],
            out_specs=pl.BlockSpec((rows, w), lambda i, *_: (i, 0)),
        ),
        out_shape=jax.ShapeDtypeStruct((n * rows, w), x0.dtype),
        compiler_params=pltpu.CompilerParams(
            dimension_semantics=("arbitrary",),
        ),
    )(t, sqrt_alphas_cumprod, sqrt_one_minus_alphas_cumprod, x2, n2)
    return out.reshape(n, c, h, w)
